# baseline (speedup 1.0000x reference)
import jax
import jax.numpy as jnp
from jax import lax
from jax.experimental import pallas as pl
from jax.experimental.pallas import tpu as pltpu
from jax.experimental.pallas import tpu_sc as plsc

L = 16
V = 8192
NROWS = 512
NWORKERS = 32
ROWS_PER = NROWS // NWORKERS
CHI = 4
NCI = ROWS_PER // CHI
NBUF = 3
NCHAIN = 4
NSTEP = V // (L * NCHAIN)


def _merge(ma, ia, mb, ib):
    take = (mb > ma) | ((mb == ma) & (ib < ia))
    return jnp.where(take, mb, ma), jnp.where(take, ib, ia)


def _argmax_row(xbuf, r, lanes):
    ms = [jnp.full((L,), -jnp.inf, jnp.float32) for _ in range(NCHAIN)]
    steps = [jnp.zeros((L,), jnp.int32) for _ in range(NCHAIN)]

    def cbody(j, carry):
        ms, steps = carry
        base = j * (L * NCHAIN)
        jv = jnp.full((L,), j, jnp.int32)
        nms, nsteps = [], []
        for k in range(NCHAIN):
            x = xbuf[r, pl.ds(base + k * L, L)]
            cond = x > ms[k]
            nms.append(jnp.maximum(x, ms[k]))
            nsteps.append(jnp.where(cond, jv, steps[k]))
        return (tuple(nms), tuple(nsteps))

    ms, steps = lax.fori_loop(
        0, NSTEP, cbody, (tuple(ms), tuple(steps)), unroll=2
    )

    iis = [
        steps[k] * (L * NCHAIN) + (lanes + L * k) for k in range(NCHAIN)
    ]
    m01, i01 = _merge(ms[0], iis[0], ms[1], iis[1])
    m23, i23 = _merge(ms[2], iis[2], ms[3], iis[3])
    m, idx = _merge(m01, i01, m23, i23)

    gm = m[0]
    gi = idx[0]
    for k in range(1, L):
        mv = m[k]
        iv = idx[k]
        take = (mv > gm) | ((mv == gm) & (iv < gi))
        gm = jnp.where(take, mv, gm)
        gi = jnp.where(take, iv, gi)
    return gi


def _body(x_hbm, out_hbm, xb0, xb1, xb2, ob0, ob1, si0, si1, si2, so0, so1):
    cid = lax.axis_index("c")
    sid = lax.axis_index("s")
    wid = sid * 2 + cid
    base = wid * ROWS_PER

    xbufs = (xb0, xb1, xb2)
    obufs = (ob0, ob1)
    sins = (si0, si1, si2)
    souts = (so0, so1)

    lanes = lax.iota(jnp.int32, L)
    zeros = jnp.zeros((L,), jnp.float32)
    ones = jnp.ones((L,), jnp.float32)
    mask0 = lanes == 0
    row0 = jnp.zeros((L,), jnp.int32)

    def zbody(j, c):
        ob0[0, pl.ds(j * L, L)] = zeros
        ob1[0, pl.ds(j * L, L)] = zeros
        return c

    lax.fori_loop(0, V // L, zbody, 0)

    for p in range(NBUF - 1):
        pltpu.async_copy(
            x_hbm.at[pl.ds(base + p * CHI, CHI)], xbufs[p], sins[p]
        )

    prev = [None, None]
    for ci in range(NCI):
        slot = ci % NBUF
        crow = base + ci * CHI
        pltpu.make_async_copy(
            x_hbm.at[pl.ds(crow, CHI)], xbufs[slot], sins[slot]
        ).wait()
        if ci + NBUF - 1 < NCI:
            nslot = (ci + NBUF - 1) % NBUF
            pltpu.async_copy(
                x_hbm.at[pl.ds(crow + (NBUF - 1) * CHI, CHI)],
                xbufs[nslot],
                sins[nslot],
            )

        for r in range(CHI):
            gr = ci * CHI + r
            gi = _argmax_row(xbufs[slot], r, lanes)
            idxv = jnp.full((L,), gi, jnp.int32)

            oslot = gr % 2
            if gr >= 2:
                pltpu.make_async_copy(
                    obufs[oslot],
                    out_hbm.at[pl.ds(base + gr - 2, 1)],
                    souts[oslot],
                ).wait()
                plsc.store_scatter(
                    obufs[oslot], [row0, prev[oslot]], zeros, mask=mask0
                )

            plsc.store_scatter(obufs[oslot], [row0, idxv], ones, mask=mask0)
            pltpu.async_copy(
                obufs[oslot], out_hbm.at[pl.ds(base + gr, 1)], souts[oslot]
            )
            prev[oslot] = idxv

    pltpu.make_async_copy(
        ob0, out_hbm.at[pl.ds(base + ROWS_PER - 2, 1)], so0
    ).wait()
    pltpu.make_async_copy(
        ob1, out_hbm.at[pl.ds(base + ROWS_PER - 1, 1)], so1
    ).wait()


@jax.jit
def kernel(logits):
    B, S, _ = logits.shape
    x = logits.reshape(NROWS, V)
    out = pl.kernel(
        _body,
        out_type=jax.ShapeDtypeStruct((NROWS, V), jnp.float32),
        mesh=plsc.VectorSubcoreMesh(core_axis_name="c", subcore_axis_name="s"),
        compiler_params=pltpu.CompilerParams(needs_layout_passes=False),
        scratch_types=[
            pltpu.VMEM((CHI, V), jnp.float32),
            pltpu.VMEM((CHI, V), jnp.float32),
            pltpu.VMEM((CHI, V), jnp.float32),
            pltpu.VMEM((1, V), jnp.float32),
            pltpu.VMEM((1, V), jnp.float32),
            pltpu.SemaphoreType.DMA,
            pltpu.SemaphoreType.DMA,
            pltpu.SemaphoreType.DMA,
            pltpu.SemaphoreType.DMA,
            pltpu.SemaphoreType.DMA,
        ],
    )(x)
    return out.reshape(B, S, V)


# speedup vs baseline: 1.0579x; 1.0579x over previous
import jax
import jax.numpy as jnp
from jax import lax
from jax.experimental import pallas as pl
from jax.experimental.pallas import tpu as pltpu
from jax.experimental.pallas import tpu_sc as plsc

L = 16
V = 8192
NROWS = 512
NWORKERS = 32
ROWS_PER = NROWS // NWORKERS
CH = 2
NCHUNKS = ROWS_PER // CH
NBUF = 4
NCHAIN = 4
NSTEP = V // (L * NCHAIN)


def _merge(ma, ia, mb, ib):
    take = (mb > ma) | ((mb == ma) & (ib < ia))
    return jnp.where(take, mb, ma), jnp.where(take, ib, ia)


def _argmax_row(xbuf, r, lanes):
    ms = [jnp.full((L,), -jnp.inf, jnp.float32) for _ in range(NCHAIN)]
    steps = [jnp.zeros((L,), jnp.int32) for _ in range(NCHAIN)]

    def cbody(j, carry):
        ms, steps = carry
        base = j * (L * NCHAIN)
        jv = jnp.full((L,), j, jnp.int32)
        nms, nsteps = [], []
        for k in range(NCHAIN):
            x = xbuf[r, pl.ds(base + k * L, L)]
            cond = x > ms[k]
            nms.append(jnp.maximum(x, ms[k]))
            nsteps.append(jnp.where(cond, jv, steps[k]))
        return (tuple(nms), tuple(nsteps))

    ms, steps = lax.fori_loop(
        0, NSTEP, cbody, (tuple(ms), tuple(steps)), unroll=2
    )

    iis = [
        steps[k] * (L * NCHAIN) + (lanes + L * k) for k in range(NCHAIN)
    ]
    m01, i01 = _merge(ms[0], iis[0], ms[1], iis[1])
    m23, i23 = _merge(ms[2], iis[2], ms[3], iis[3])
    m, idx = _merge(m01, i01, m23, i23)

    gm = m[0]
    gi = idx[0]
    for k in range(1, L):
        mv = m[k]
        iv = idx[k]
        take = (mv > gm) | ((mv == gm) & (iv < gi))
        gm = jnp.where(take, mv, gm)
        gi = jnp.where(take, iv, gi)
    return gi


def _body(x_hbm, out_hbm, xb0, xb1, xb2, xb3, ob0, ob1, si0, si1, si2, si3, so0, so1):
    cid = lax.axis_index("c")
    sid = lax.axis_index("s")
    wid = sid * 2 + cid
    base = wid * ROWS_PER

    xbufs = (xb0, xb1, xb2, xb3)
    obufs = (ob0, ob1)
    sins = (si0, si1, si2, si3)
    souts = (so0, so1)

    lanes = lax.iota(jnp.int32, L)
    zeros = jnp.zeros((L,), jnp.float32)
    ones = jnp.ones((L,), jnp.float32)
    mask0 = lanes == 0

    def zbody(j, c):
        for r in range(CH):
            ob0[r, pl.ds(j * L, L)] = zeros
            ob1[r, pl.ds(j * L, L)] = zeros
        return c

    lax.fori_loop(0, V // L, zbody, 0)

    for p in range(NBUF - 1):
        pltpu.async_copy(
            x_hbm.at[pl.ds(base + p * CH, CH)], xbufs[p], sins[p]
        )

    prev = [None, None]
    for c in range(NCHUNKS):
        slot = c % NBUF
        row = base + c * CH
        pltpu.make_async_copy(
            x_hbm.at[pl.ds(row, CH)], xbufs[slot], sins[slot]
        ).wait()
        if c + NBUF - 1 < NCHUNKS:
            nslot = (c + NBUF - 1) % NBUF
            pltpu.async_copy(
                x_hbm.at[pl.ds(row + (NBUF - 1) * CH, CH)],
                xbufs[nslot],
                sins[nslot],
            )

        idxvs = []
        for r in range(CH):
            gi = _argmax_row(xbufs[slot], r, lanes)
            idxvs.append(
                (jnp.full((L,), r, jnp.int32), jnp.full((L,), gi, jnp.int32))
            )

        oslot = c % 2
        if c >= 2:
            prow = base + (c - 2) * CH
            pltpu.make_async_copy(
                obufs[oslot], out_hbm.at[pl.ds(prow, CH)], souts[oslot]
            ).wait()
            for r in range(CH):
                plsc.store_scatter(
                    obufs[oslot], list(prev[oslot][r]), zeros, mask=mask0
                )

        for r in range(CH):
            plsc.store_scatter(obufs[oslot], list(idxvs[r]), ones, mask=mask0)
        pltpu.async_copy(obufs[oslot], out_hbm.at[pl.ds(row, CH)], souts[oslot])
        prev[oslot] = idxvs

    pltpu.make_async_copy(
        ob0, out_hbm.at[pl.ds(base + (NCHUNKS - 2) * CH, CH)], so0
    ).wait()
    pltpu.make_async_copy(
        ob1, out_hbm.at[pl.ds(base + (NCHUNKS - 1) * CH, CH)], so1
    ).wait()


@jax.jit
def kernel(logits):
    B, S, _ = logits.shape
    x = logits.reshape(NROWS, V)
    out = pl.kernel(
        _body,
        out_type=jax.ShapeDtypeStruct((NROWS, V), jnp.float32),
        mesh=plsc.VectorSubcoreMesh(core_axis_name="c", subcore_axis_name="s"),
        compiler_params=pltpu.CompilerParams(needs_layout_passes=False),
        scratch_types=[
            pltpu.VMEM((CH, V), jnp.float32),
            pltpu.VMEM((CH, V), jnp.float32),
            pltpu.VMEM((CH, V), jnp.float32),
            pltpu.VMEM((CH, V), jnp.float32),
            pltpu.VMEM((CH, V), jnp.float32),
            pltpu.VMEM((CH, V), jnp.float32),
            pltpu.SemaphoreType.DMA,
            pltpu.SemaphoreType.DMA,
            pltpu.SemaphoreType.DMA,
            pltpu.SemaphoreType.DMA,
            pltpu.SemaphoreType.DMA,
            pltpu.SemaphoreType.DMA,
        ],
    )(x)
    return out.reshape(B, S, V)
